# Initial kernel scaffold; baseline (speedup 1.0000x reference)
import jax
import jax.numpy as jnp
from jax.experimental import pallas as pl


def kernel(x, edge_index, W1_rel, b1, W1_root, W2_rel, b2, W2_root, W3, b3):
    raise NotImplementedError("write your pallas kernel here")


# speedup vs baseline: 7.8705x; 7.8705x over previous
import functools

import jax
import jax.numpy as jnp
from jax import lax
from jax.experimental import pallas as pl
from jax.experimental.pallas import tpu as pltpu
from jax.experimental.pallas import tpu_sc as plsc

_N = 10000
_D = 128
_H = 128
_OUT = 192
_E = 320000

_NC = 2
_NS = 16
_NW = _NC * _NS
_EPT = _E // _NW
_CHUNK = 125
_CHUNKS = _EPT // _CHUNK
_ZROWS = 400
_ZTASKS = _N // _ZROWS

def _seg_sum_body(x_hbm, src_hbm, dst_hbm, zblk_hbm, out_hbm,
                  acc, src_v, dst_v, rows_v, sem):
    cid = lax.axis_index("c")
    sid = lax.axis_index("s")
    wid = cid * _NS + sid

    pltpu.sync_copy(zblk_hbm, acc.at[pl.ds(sid * _ZROWS, _ZROWS)])

    @pl.when(sid < _ZTASKS - _NS)
    def _():
        pltpu.sync_copy(zblk_hbm,
                        acc.at[pl.ds((sid + _NS) * _ZROWS, _ZROWS)])

    pltpu.sync_copy(src_hbm.at[wid], src_v)
    pltpu.sync_copy(dst_hbm.at[wid], dst_v)
    plsc.subcore_barrier()

    def body(j, carry):
        pltpu.async_copy(x_hbm.at[src_v.at[j]], rows_v, sem).wait()
        pltpu.sync_copy(rows_v, acc.at[dst_v.at[j]], add=True)
        return carry

    lax.fori_loop(0, _CHUNKS, body, 0)
    plsc.subcore_barrier()

    pltpu.sync_copy(acc.at[pl.ds(sid * _ZROWS, _ZROWS)],
                    out_hbm.at[cid, pl.ds(sid * _ZROWS, _ZROWS)])

    @pl.when(sid < _ZTASKS - _NS)
    def _():
        pltpu.sync_copy(acc.at[pl.ds((sid + _NS) * _ZROWS, _ZROWS)],
                        out_hbm.at[cid, pl.ds((sid + _NS) * _ZROWS, _ZROWS)])


@functools.cache
def _seg_sum():
    mesh = plsc.VectorSubcoreMesh(core_axis_name="c", subcore_axis_name="s",
                                  num_cores=_NC, num_subcores=_NS)
    return pl.kernel(
        _seg_sum_body,
        out_type=jax.ShapeDtypeStruct((_NC, _N, _D), jnp.float32),
        mesh=mesh,
        scratch_types=[
            pltpu.VMEM_SHARED((_N, _D), jnp.float32),
            pltpu.VMEM((_CHUNKS, _CHUNK), jnp.int32),
            pltpu.VMEM((_CHUNKS, _CHUNK), jnp.int32),
            pltpu.VMEM((_CHUNK, _D), jnp.float32),
            pltpu.SemaphoreType.DMA,
        ],
    )


_BLK = 1000


def _lin1_body(acc0_ref, acc1_ref, x_ref, wrel_ref, wroot_ref, b_ref, o_ref):
    agg = acc0_ref[...] + acc1_ref[...]
    o_ref[...] = jnp.maximum(
        jnp.dot(agg, wrel_ref[...], preferred_element_type=jnp.float32)
        + jnp.dot(x_ref[...], wroot_ref[...], preferred_element_type=jnp.float32)
        + b_ref[...],
        0.0,
    )


def _lin2_body(acc0_ref, acc1_ref, h1_ref, wrel_ref, wroot_ref, b_ref,
               w3_ref, b3_ref, o_ref):
    agg = acc0_ref[...] + acc1_ref[...]
    h2 = jnp.maximum(
        jnp.dot(agg, wrel_ref[...], preferred_element_type=jnp.float32)
        + jnp.dot(h1_ref[...], wroot_ref[...], preferred_element_type=jnp.float32)
        + b_ref[...],
        0.0,
    )
    o_ref[...] = (
        jnp.dot(h2, w3_ref[...], preferred_element_type=jnp.float32)
        + b3_ref[...]
    )


def _row_spec(i):
    return (i, 0)


def _rep_spec(i):
    return (0, 0)


def _lin1(acc0, acc1, x, wrel, wroot, b):
    return pl.pallas_call(
        _lin1_body,
        grid=(_N // _BLK,),
        in_specs=[
            pl.BlockSpec((_BLK, _D), _row_spec),
            pl.BlockSpec((_BLK, _D), _row_spec),
            pl.BlockSpec((_BLK, _D), _row_spec),
            pl.BlockSpec((_D, _H), _rep_spec),
            pl.BlockSpec((_D, _H), _rep_spec),
            pl.BlockSpec((1, _H), _rep_spec),
        ],
        out_specs=pl.BlockSpec((_BLK, _H), _row_spec),
        out_shape=jax.ShapeDtypeStruct((_N, _H), jnp.float32),
    )(acc0, acc1, x, wrel, wroot, b)


def _lin2(acc0, acc1, h1, wrel, wroot, b, w3, b3):
    return pl.pallas_call(
        _lin2_body,
        grid=(_N // _BLK,),
        in_specs=[
            pl.BlockSpec((_BLK, _H), _row_spec),
            pl.BlockSpec((_BLK, _H), _row_spec),
            pl.BlockSpec((_BLK, _H), _row_spec),
            pl.BlockSpec((_H, _H), _rep_spec),
            pl.BlockSpec((_H, _H), _rep_spec),
            pl.BlockSpec((1, _H), _rep_spec),
            pl.BlockSpec((_H, _OUT), _rep_spec),
            pl.BlockSpec((1, _OUT), _rep_spec),
        ],
        out_specs=pl.BlockSpec((_BLK, _OUT), _row_spec),
        out_shape=jax.ShapeDtypeStruct((_N, _OUT), jnp.float32),
    )(acc0, acc1, h1, wrel, wroot, b, w3, b3)


def kernel(x, edge_index, W1_rel, b1, W1_root, W2_rel, b2, W2_root, W3, b3):
    src = edge_index[0].reshape(_NW, _CHUNKS, _CHUNK)
    dst = edge_index[1].reshape(_NW, _CHUNKS, _CHUNK)
    zblk = jnp.zeros((_ZROWS, _D), jnp.float32)

    seg = _seg_sum()
    acc1 = seg(x, src, dst, zblk)
    h1 = _lin1(acc1[0], acc1[1], x, W1_rel, W1_root, b1.reshape(1, _H))
    acc2 = seg(h1, src, dst, zblk)
    return _lin2(acc2[0], acc2[1], h1, W2_rel, W2_root, b2.reshape(1, _H),
                 W3, b3.reshape(1, _OUT))


# speedup vs baseline: 11.2252x; 1.4262x over previous
import functools

import jax
import jax.numpy as jnp
from jax import lax
from jax.experimental import pallas as pl
from jax.experimental.pallas import tpu as pltpu
from jax.experimental.pallas import tpu_sc as plsc

_N = 10000
_D = 128
_H = 128
_OUT = 192
_E = 320000

_NC = 2
_NS = 16
_NW = _NC * _NS
_EPT = _E // _NW
_CHUNK = 125
_CHUNKS = _EPT // _CHUNK
_HCH = _CHUNKS // 2
_ZROWS = 400
_ZTASKS = _N // _ZROWS

def _seg_sum_body(x_hbm, src_hbm, dst_hbm, zblk_hbm, out_hbm,
                  acc, src_v, dst_v, rows_a, rows_b, sem_a, sem_b):
    cid = lax.axis_index("c")
    sid = lax.axis_index("s")
    wid = cid * _NS + sid

    pltpu.sync_copy(zblk_hbm, acc.at[pl.ds(sid * _ZROWS, _ZROWS)])

    @pl.when(sid < _ZTASKS - _NS)
    def _():
        pltpu.sync_copy(zblk_hbm,
                        acc.at[pl.ds((sid + _NS) * _ZROWS, _ZROWS)])

    plsc.subcore_barrier()

    for h in range(2):
        pltpu.sync_copy(src_hbm.at[wid, pl.ds(h * _HCH, _HCH)], src_v)
        pltpu.sync_copy(dst_hbm.at[wid, pl.ds(h * _HCH, _HCH)], dst_v)
        pltpu.async_copy(x_hbm.at[src_v.at[0]], rows_a, sem_a)

        def body(i, carry):
            j = i * 2
            pltpu.async_copy(x_hbm.at[src_v.at[j + 1]], rows_b, sem_b)
            pltpu.make_async_copy(x_hbm.at[src_v.at[j]], rows_a, sem_a).wait()
            pltpu.sync_copy(rows_a, acc.at[dst_v.at[j]], add=True)

            @pl.when(j + 2 < _HCH)
            def _():
                pltpu.async_copy(x_hbm.at[src_v.at[j + 2]], rows_a, sem_a)

            pltpu.make_async_copy(x_hbm.at[src_v.at[j + 1]], rows_b,
                                  sem_b).wait()
            pltpu.sync_copy(rows_b, acc.at[dst_v.at[j + 1]], add=True)
            return carry

        lax.fori_loop(0, _HCH // 2, body, 0)

    plsc.subcore_barrier()

    pltpu.sync_copy(acc.at[pl.ds(sid * _ZROWS, _ZROWS)],
                    out_hbm.at[cid, pl.ds(sid * _ZROWS, _ZROWS)])

    @pl.when(sid < _ZTASKS - _NS)
    def _():
        pltpu.sync_copy(acc.at[pl.ds((sid + _NS) * _ZROWS, _ZROWS)],
                        out_hbm.at[cid, pl.ds((sid + _NS) * _ZROWS, _ZROWS)])


@functools.cache
def _seg_sum():
    mesh = plsc.VectorSubcoreMesh(core_axis_name="c", subcore_axis_name="s",
                                  num_cores=_NC, num_subcores=_NS)
    return pl.kernel(
        _seg_sum_body,
        out_type=jax.ShapeDtypeStruct((_NC, _N, _D), jnp.float32),
        mesh=mesh,
        scratch_types=[
            pltpu.VMEM_SHARED((_N, _D), jnp.float32),
            pltpu.VMEM((_HCH, _CHUNK), jnp.int32),
            pltpu.VMEM((_HCH, _CHUNK), jnp.int32),
            pltpu.VMEM((_CHUNK, _D), jnp.float32),
            pltpu.VMEM((_CHUNK, _D), jnp.float32),
            pltpu.SemaphoreType.DMA,
            pltpu.SemaphoreType.DMA,
        ],
    )


_BLK = 1000


def _lin1_body(acc0_ref, acc1_ref, x_ref, wrel_ref, wroot_ref, b_ref, o_ref):
    agg = acc0_ref[...] + acc1_ref[...]
    o_ref[...] = jnp.maximum(
        jnp.dot(agg, wrel_ref[...], preferred_element_type=jnp.float32)
        + jnp.dot(x_ref[...], wroot_ref[...], preferred_element_type=jnp.float32)
        + b_ref[...],
        0.0,
    )


def _lin2_body(acc0_ref, acc1_ref, h1_ref, wrel_ref, wroot_ref, b_ref,
               w3_ref, b3_ref, o_ref):
    agg = acc0_ref[...] + acc1_ref[...]
    h2 = jnp.maximum(
        jnp.dot(agg, wrel_ref[...], preferred_element_type=jnp.float32)
        + jnp.dot(h1_ref[...], wroot_ref[...], preferred_element_type=jnp.float32)
        + b_ref[...],
        0.0,
    )
    o_ref[...] = (
        jnp.dot(h2, w3_ref[...], preferred_element_type=jnp.float32)
        + b3_ref[...]
    )


def _row_spec(i):
    return (i, 0)


def _rep_spec(i):
    return (0, 0)


def _lin1(acc0, acc1, x, wrel, wroot, b):
    return pl.pallas_call(
        _lin1_body,
        grid=(_N // _BLK,),
        in_specs=[
            pl.BlockSpec((_BLK, _D), _row_spec),
            pl.BlockSpec((_BLK, _D), _row_spec),
            pl.BlockSpec((_BLK, _D), _row_spec),
            pl.BlockSpec((_D, _H), _rep_spec),
            pl.BlockSpec((_D, _H), _rep_spec),
            pl.BlockSpec((1, _H), _rep_spec),
        ],
        out_specs=pl.BlockSpec((_BLK, _H), _row_spec),
        out_shape=jax.ShapeDtypeStruct((_N, _H), jnp.float32),
    )(acc0, acc1, x, wrel, wroot, b)


def _lin2(acc0, acc1, h1, wrel, wroot, b, w3, b3):
    return pl.pallas_call(
        _lin2_body,
        grid=(_N // _BLK,),
        in_specs=[
            pl.BlockSpec((_BLK, _H), _row_spec),
            pl.BlockSpec((_BLK, _H), _row_spec),
            pl.BlockSpec((_BLK, _H), _row_spec),
            pl.BlockSpec((_H, _H), _rep_spec),
            pl.BlockSpec((_H, _H), _rep_spec),
            pl.BlockSpec((1, _H), _rep_spec),
            pl.BlockSpec((_H, _OUT), _rep_spec),
            pl.BlockSpec((1, _OUT), _rep_spec),
        ],
        out_specs=pl.BlockSpec((_BLK, _OUT), _row_spec),
        out_shape=jax.ShapeDtypeStruct((_N, _OUT), jnp.float32),
    )(acc0, acc1, h1, wrel, wroot, b, w3, b3)


def kernel(x, edge_index, W1_rel, b1, W1_root, W2_rel, b2, W2_root, W3, b3):
    src = edge_index[0].reshape(_NW, _CHUNKS, _CHUNK)
    dst = edge_index[1].reshape(_NW, _CHUNKS, _CHUNK)
    zblk = jnp.zeros((_ZROWS, _D), jnp.float32)

    seg = _seg_sum()
    acc1 = seg(x, src, dst, zblk)
    h1 = _lin1(acc1[0], acc1[1], x, W1_rel, W1_root, b1.reshape(1, _H))
    acc2 = seg(h1, src, dst, zblk)
    return _lin2(acc2[0], acc2[1], h1, W2_rel, W2_root, b2.reshape(1, _H),
                 W3, b3.reshape(1, _OUT))
